# baseline (speedup 1.0000x reference)
import jax
import jax.numpy as jnp
from jax import lax
from jax.experimental import pallas as pl
from jax.experimental.pallas import tpu as pltpu

_MAX_MASK_RATIO = 0.1
_T_BLK = 1024

_MASK_CACHE = {}


def _static_mask_bounds(B, D):
    if (B, D) not in _MASK_CACHE:
        max_mask_len = int(D * _MAX_MASK_RATIO)
        with jax.ensure_compile_time_eval():
            key = jax.random.key(42)
            k1, k2 = jax.random.split(key)
            mask_len = jax.random.randint(k1, (B,), 1, max_mask_len + 1)
            mask_start = jax.random.randint(k2, (B,), 0, D - max_mask_len + 1)
            starts = [int(x) for x in mask_start]
            ends = [int(s + l) for s, l in zip(starts, [int(x) for x in mask_len])]
        _MASK_CACHE[(B, D)] = list(zip(starts, ends))
    return _MASK_CACHE[(B, D)]


def kernel(mean):
    B, T, D = mean.shape
    bounds = _static_mask_bounds(B, D)
    starts = jnp.array([s for s, _ in bounds], dtype=jnp.int32)
    ends = jnp.array([e for _, e in bounds], dtype=jnp.int32)

    def body(starts_ref, ends_ref, x_ref, o_ref):
        b = pl.program_id(0)
        s = starts_ref[b]
        e = ends_ref[b]
        col = lax.broadcasted_iota(jnp.int32, (_T_BLK, D), 1)
        keep = (col < s) | (col >= e)
        o_ref[0] = jnp.where(keep, x_ref[0], 0.0)

    return pl.pallas_call(
        body,
        grid=(B, T // _T_BLK),
        in_specs=[
            pl.BlockSpec(memory_space=pltpu.SMEM),
            pl.BlockSpec(memory_space=pltpu.SMEM),
            pl.BlockSpec((1, _T_BLK, D), lambda b, t: (b, t, 0)),
        ],
        out_specs=pl.BlockSpec((1, _T_BLK, D), lambda b, t: (b, t, 0)),
        out_shape=jax.ShapeDtypeStruct((B, T, D), mean.dtype),
    )(starts, ends, mean)
